# baseline (speedup 1.0000x reference)
import functools
import jax
import jax.numpy as jnp
from jax import lax
from jax.experimental import pallas as pl
from jax.experimental.pallas import tpu as pltpu
from jax.experimental.pallas import tpu_sc as plsc

T = 500
DROPOUT_P = 0.1
RR = 3072

_F32 = jnp.float32


def _sc_gather(ts, u, ta, tb):
    B = ts.shape[0]
    info = plsc.get_sparse_core_info()
    NC, NS, L = info.num_cores, info.num_subcores, info.num_lanes
    NW = NC * NS
    bpw = B // NW
    V = ta.shape[0]

    mesh = plsc.VectorSubcoreMesh(core_axis_name="c", subcore_axis_name="s")

    @functools.partial(
        pl.kernel, mesh=mesh,
        out_type=[jax.ShapeDtypeStruct((B,), _F32)] * 4,
        scratch_types=[
            pltpu.VMEM((bpw,), jnp.int32), pltpu.VMEM((bpw,), _F32),
            pltpu.VMEM((bpw,), _F32), pltpu.VMEM((bpw,), _F32),
            pltpu.VMEM((bpw,), _F32), pltpu.VMEM((bpw,), _F32),
            pltpu.SemaphoreType.DMA,
        ],
    )
    def k(ts_hbm, u_hbm, ta_hbm, tb_hbm, a_hbm, b_hbm, tn_hbm, cm_hbm,
          ts_v, u_v, a_v, b_v, tn_v, cm_v, sem):
        wid = lax.axis_index("s") * NC + lax.axis_index("c")
        base = wid * bpw
        pltpu.sync_copy(ts_hbm.at[pl.ds(base, bpw)], ts_v)
        pltpu.sync_copy(u_hbm.at[pl.ds(base, bpw)], u_v)
        pltpu.async_copy(ta_hbm.at[ts_v], a_v, sem).wait()
        pltpu.async_copy(tb_hbm.at[ts_v], b_v, sem).wait()
        tsv = ts_v[...]
        tn_v[...] = tsv.astype(_F32) / T
        cm_v[...] = jnp.where(u_v[...] < DROPOUT_P, 1.0, 0.0).astype(_F32)
        pltpu.sync_copy(a_v, a_hbm.at[pl.ds(base, bpw)])
        pltpu.sync_copy(b_v, b_hbm.at[pl.ds(base, bpw)])
        pltpu.sync_copy(tn_v, tn_hbm.at[pl.ds(base, bpw)])
        pltpu.sync_copy(cm_v, cm_hbm.at[pl.ds(base, bpw)])

    return k(ts, u, ta, tb)


def _tc_body(x_ref, n_ref, a_ref, b_ref, xt_ref, nout_ref):
    B = a_ref.shape[0]
    a = a_ref[...].reshape(1, B)
    b = b_ref[...].reshape(1, B)
    n = n_ref[...]
    xt_ref[...] = a * x_ref[...] + b * n
    nout_ref[...] = n


def kernel(x, cls, timestep, noise, u, sqrt_abar_t, sqrt_abar_t1):
    B, C, H, W = x.shape
    F = C * H * W
    xv = x.transpose(1, 2, 3, 0).reshape(F, B)
    nv = noise.transpose(1, 2, 3, 0).reshape(F, B)

    a, bv, tn, cm = _sc_gather(timestep, u, sqrt_abar_t, sqrt_abar_t1)

    grid = (F // RR,)
    big = pl.BlockSpec((RR, B), lambda i: (i, 0))
    vec1 = pl.BlockSpec((B,), lambda i: (0,))

    xt, nout = pl.pallas_call(
        _tc_body,
        grid=grid,
        in_specs=[big, big, vec1, vec1],
        out_specs=[big, big],
        out_shape=[
            jax.ShapeDtypeStruct((F, B), _F32),
            jax.ShapeDtypeStruct((F, B), _F32),
        ],
    )(xv, nv, a, bv)

    img = lambda v: v.reshape(C, H, W, B).transpose(3, 0, 1, 2)
    return (img(nout), img(xt), cls, tn, cm)


# speedup vs baseline: 1.0652x; 1.0652x over previous
import functools
import jax
import jax.numpy as jnp
from jax import lax
from jax.experimental import pallas as pl
from jax.experimental.pallas import tpu as pltpu
from jax.experimental.pallas import tpu_sc as plsc

T = 500
DROPOUT_P = 0.1
TPAD = 512
RR = 3072

_F32 = jnp.float32


def _sc_per_sample(ts, u):
    B = ts.shape[0]
    info = plsc.get_sparse_core_info()
    NC, NS = info.num_cores, info.num_subcores
    bpw = B // (NC * NS)

    mesh = plsc.VectorSubcoreMesh(core_axis_name="c", subcore_axis_name="s")

    @functools.partial(
        pl.kernel, mesh=mesh,
        out_type=[jax.ShapeDtypeStruct((B,), _F32)] * 2,
        scratch_types=[
            pltpu.VMEM((bpw,), jnp.int32), pltpu.VMEM((bpw,), _F32),
            pltpu.VMEM((bpw,), _F32), pltpu.VMEM((bpw,), _F32),
        ],
    )
    def k(ts_hbm, u_hbm, tn_hbm, cm_hbm, ts_v, u_v, tn_v, cm_v):
        wid = lax.axis_index("s") * NC + lax.axis_index("c")
        base = wid * bpw
        pltpu.sync_copy(ts_hbm.at[pl.ds(base, bpw)], ts_v)
        pltpu.sync_copy(u_hbm.at[pl.ds(base, bpw)], u_v)
        tn_v[...] = ts_v[...].astype(_F32) / T
        cm_v[...] = jnp.where(u_v[...] < DROPOUT_P, 1.0, 0.0).astype(_F32)
        pltpu.sync_copy(tn_v, tn_hbm.at[pl.ds(base, bpw)])
        pltpu.sync_copy(cm_v, cm_hbm.at[pl.ds(base, bpw)])

    return k(ts, u)


def _tc_body(x_ref, n_ref, ts_ref, ta_ref, tb_ref,
             xt_ref, nout_ref, a_ref, b_ref):
    i = pl.program_id(0)

    @pl.when(i == 0)
    def _():
        B = ts_ref.shape[0]
        ts = ts_ref[...].reshape(1, B)
        row = jax.lax.broadcasted_iota(jnp.int32, (TPAD, B), 0)
        onehot = row == ts
        a_ref[...] = jnp.sum(jnp.where(onehot, ta_ref[...], 0.0),
                             axis=0, keepdims=True)
        b_ref[...] = jnp.sum(jnp.where(onehot, tb_ref[...], 0.0),
                             axis=0, keepdims=True)

    n = n_ref[...]
    xt_ref[...] = a_ref[...] * x_ref[...] + b_ref[...] * n
    nout_ref[...] = n


def kernel(x, cls, timestep, noise, u, sqrt_abar_t, sqrt_abar_t1):
    B, C, H, W = x.shape
    F = C * H * W
    xv = x.transpose(1, 2, 3, 0).reshape(F, B)
    nv = noise.transpose(1, 2, 3, 0).reshape(F, B)
    ta = jnp.zeros((TPAD, 1), _F32).at[:T, 0].set(sqrt_abar_t)
    tb = jnp.zeros((TPAD, 1), _F32).at[:T, 0].set(sqrt_abar_t1)

    tn, cm = _sc_per_sample(timestep, u)

    grid = (F // RR,)
    big = pl.BlockSpec((RR, B), lambda i: (i, 0))
    vec1 = pl.BlockSpec((B,), lambda i: (0,))
    tab = pl.BlockSpec((TPAD, 1), lambda i: (0, 0))

    xt, nout = pl.pallas_call(
        _tc_body,
        grid=grid,
        in_specs=[big, big, vec1, tab, tab],
        out_specs=[big, big],
        out_shape=[
            jax.ShapeDtypeStruct((F, B), _F32),
            jax.ShapeDtypeStruct((F, B), _F32),
        ],
        scratch_shapes=[pltpu.VMEM((1, B), _F32),
                        pltpu.VMEM((1, B), _F32)],
    )(xv, nv, timestep, ta, tb)

    img = lambda v: v.reshape(C, H, W, B).transpose(3, 0, 1, 2)
    return (img(nout), img(xt), cls, tn, cm)


# speedup vs baseline: 1.6796x; 1.5768x over previous
import jax
import jax.numpy as jnp
from jax.experimental import pallas as pl
from jax.experimental.pallas import tpu as pltpu

T = 500
DROPOUT_P = 0.1
RR = 3072


def _ddpm_body(x_ref, n_ref, ts_ref, u_ref, ta_ref, tb_ref,
               xt_ref, nout_ref, tn_ref, cm_ref, a_ref, b_ref):
    i = pl.program_id(0)

    @pl.when(i == 0)
    def _():
        B = ts_ref.shape[0]
        ts = ts_ref[...].reshape(1, B)
        row = jax.lax.broadcasted_iota(jnp.int32, (T, B), 0)
        onehot = (row == ts).astype(jnp.float32)
        tabs = jnp.concatenate([ta_ref[...].reshape(1, T),
                                tb_ref[...].reshape(1, T)], axis=0)
        ab = jax.lax.dot_general(tabs, onehot, (((1,), (0,)), ((), ())),
                                 preferred_element_type=jnp.float32)
        a_ref[...] = ab[0:1, :]
        b_ref[...] = ab[1:2, :]
        tn_ref[...] = ts.astype(jnp.float32).reshape(B) / T
        cm_ref[...] = (u_ref[...] < DROPOUT_P).astype(jnp.float32)

    n = n_ref[...]
    xt_ref[...] = a_ref[...] * x_ref[...] + b_ref[...] * n
    nout_ref[...] = n


def kernel(x, cls, timestep, noise, u, sqrt_abar_t, sqrt_abar_t1):
    B, C, H, W = x.shape
    F = C * H * W
    xv = x.transpose(1, 2, 3, 0).reshape(F, B)
    nv = noise.transpose(1, 2, 3, 0).reshape(F, B)

    grid = (F // RR,)
    big = pl.BlockSpec((RR, B), lambda i: (i, 0))
    vecB = pl.BlockSpec((B,), lambda i: (0,))
    vecT = pl.BlockSpec((T,), lambda i: (0,))

    xt, nout, tn, cm = pl.pallas_call(
        _ddpm_body,
        grid=grid,
        in_specs=[big, big, vecB, vecB, vecT, vecT],
        out_specs=[big, big, vecB, vecB],
        out_shape=[
            jax.ShapeDtypeStruct((F, B), jnp.float32),
            jax.ShapeDtypeStruct((F, B), jnp.float32),
            jax.ShapeDtypeStruct((B,), jnp.float32),
            jax.ShapeDtypeStruct((B,), jnp.float32),
        ],
        scratch_shapes=[pltpu.VMEM((1, B), jnp.float32),
                        pltpu.VMEM((1, B), jnp.float32)],
    )(xv, nv, timestep, u, sqrt_abar_t, sqrt_abar_t1)

    img = lambda v: v.reshape(C, H, W, B).transpose(3, 0, 1, 2)
    return (img(nout), img(xt), cls, tn, cm)


# speedup vs baseline: 1.6915x; 1.0071x over previous
import jax
import jax.numpy as jnp
from jax.experimental import pallas as pl
from jax.experimental.pallas import tpu as pltpu

T = 500
DROPOUT_P = 0.1
RR = 3072


def _ddpm_body(x_ref, n_ref, ts_ref, u_ref, ta_ref, tb_ref,
               xt_ref, nout_ref, tn_ref, cm_ref, a_ref, b_ref):
    i = pl.program_id(0)

    @pl.when(i == 0)
    def _():
        B = ts_ref.shape[0]
        ts = ts_ref[...].reshape(1, B)
        row = jax.lax.broadcasted_iota(jnp.int32, (T, B), 0)
        onehot = row == ts
        tat = ta_ref[...].reshape(1, T).T
        tbt = tb_ref[...].reshape(1, T).T
        a_ref[...] = jnp.sum(jnp.where(onehot, tat, 0.0),
                             axis=0, keepdims=True)
        b_ref[...] = jnp.sum(jnp.where(onehot, tbt, 0.0),
                             axis=0, keepdims=True)
        tn_ref[...] = ts.astype(jnp.float32).reshape(B) / T
        cm_ref[...] = (u_ref[...] < DROPOUT_P).astype(jnp.float32)

    n = n_ref[...]
    xt_ref[...] = a_ref[...] * x_ref[...] + b_ref[...] * n
    nout_ref[...] = n


def kernel(x, cls, timestep, noise, u, sqrt_abar_t, sqrt_abar_t1):
    B, C, H, W = x.shape
    F = C * H * W
    xv = x.transpose(1, 2, 3, 0).reshape(F, B)
    nv = noise.transpose(1, 2, 3, 0).reshape(F, B)

    grid = (F // RR,)
    big = pl.BlockSpec((RR, B), lambda i: (i, 0))
    vecB = pl.BlockSpec((B,), lambda i: (0,))
    vecT = pl.BlockSpec((T,), lambda i: (0,))

    xt, nout, tn, cm = pl.pallas_call(
        _ddpm_body,
        grid=grid,
        in_specs=[big, big, vecB, vecB, vecT, vecT],
        out_specs=[big, big, vecB, vecB],
        out_shape=[
            jax.ShapeDtypeStruct((F, B), jnp.float32),
            jax.ShapeDtypeStruct((F, B), jnp.float32),
            jax.ShapeDtypeStruct((B,), jnp.float32),
            jax.ShapeDtypeStruct((B,), jnp.float32),
        ],
        scratch_shapes=[pltpu.VMEM((1, B), jnp.float32),
                        pltpu.VMEM((1, B), jnp.float32)],
    )(xv, nv, timestep, u, sqrt_abar_t, sqrt_abar_t1)

    img = lambda v: v.reshape(C, H, W, B).transpose(3, 0, 1, 2)
    return (img(nout), img(xt), cls, tn, cm)
